# baseline (speedup 1.0000x reference)
import jax
import jax.numpy as jnp
from jax.experimental import pallas as pl
from jax.experimental.pallas import tpu as pltpu

D_MODEL = 128
MAX_LEN = 200
SEQ = 200

_REP = 16
_K = 8


def _bcast_dma_kernel(w_ref, o_ref, scratch, sems):
    scratch[...] = jnp.broadcast_to(w_ref[...][None, :, :], scratch.shape)
    n = o_ref.shape[0] // _REP

    def _copy(i, slot):
        return pltpu.make_async_copy(
            scratch, o_ref.at[pl.ds(i * _REP, _REP)], sems.at[slot]
        )

    def _step(i, carry):
        @pl.when(i < n)
        def _start():
            _copy(i, jax.lax.rem(i, _K)).start()

        @pl.when(i >= _K)
        def _wait():
            j = i - _K
            _copy(j, jax.lax.rem(j, _K)).wait()

        return carry

    jax.lax.fori_loop(0, n + _K, _step, 0)


def kernel(x, pos_embedding_weight):
    bs, seq = x.shape
    out = pl.pallas_call(
        _bcast_dma_kernel,
        in_specs=[pl.BlockSpec(memory_space=pltpu.MemorySpace.VMEM)],
        out_specs=pl.BlockSpec(memory_space=pltpu.MemorySpace.HBM),
        out_shape=jax.ShapeDtypeStruct((bs, seq, D_MODEL), jnp.float32),
        scratch_shapes=[
            pltpu.VMEM((_REP, seq, D_MODEL), jnp.float32),
            pltpu.SemaphoreType.DMA((_K,)),
        ],
    )(pos_embedding_weight[:seq])
    return out
